# Initial kernel scaffold; baseline (speedup 1.0000x reference)
import jax
import jax.numpy as jnp
from jax.experimental import pallas as pl


def kernel(padded_char_tensor, sequence_lens, emb, Wih_f, Whh_f, bih_f, bhh_f, Wih_b, Whh_b, bih_b, bhh_b, Wp, bp, ctx):
    raise NotImplementedError("write your pallas kernel here")


# speedup vs baseline: 10.9555x; 10.9555x over previous
import functools

import jax
import jax.numpy as jnp
from jax.experimental import pallas as pl
from jax.experimental.pallas import tpu as pltpu


def _block_kernel(chars_ref, lens_ref, emb_ref, wihT_ref, whhT_f_ref,
                  whhT_b_ref, bih_ref, bhh_f_ref, bhh_b_ref, wpT_ref,
                  bp_ref, ctx_ref, out_ref):
    T, BW, _ = chars_ref.shape
    H = whhT_f_ref.shape[0]
    V = emb_ref.shape[0]

    chars = chars_ref[...]
    lens = lens_ref[...]

    iota_c = jax.lax.broadcasted_iota(jnp.int32, (T, BW, V), 2)
    oh = (chars == iota_c).astype(jnp.float32).reshape(T * BW, V)
    x = jnp.dot(oh, emb_ref[...], preferred_element_type=jnp.float32)

    xp = jnp.dot(x, wihT_ref[...], preferred_element_type=jnp.float32) \
        + bih_ref[...]
    xp_f = xp[:, :3 * H]
    xp_b = xp[:, 3 * H:]

    def cell(xt, h, whhT, bhh):
        hp = jnp.dot(h, whhT, preferred_element_type=jnp.float32) + bhh
        r = jax.nn.sigmoid(xt[:, :H] + hp[:, :H])
        z = jax.nn.sigmoid(xt[:, H:2 * H] + hp[:, H:2 * H])
        n = jnp.tanh(xt[:, 2 * H:] + r * hp[:, 2 * H:])
        return (1.0 - z) * n + z * h

    whhT_f = whhT_f_ref[...]
    whhT_b = whhT_b_ref[...]
    bhh_f = bhh_f_ref[...]
    bhh_b = bhh_b_ref[...]

    h = jnp.zeros((BW, H), jnp.float32)
    outs_f = []
    for t in range(T):
        h = cell(xp_f[t * BW:(t + 1) * BW, :], h, whhT_f, bhh_f)
        outs_f.append(h)

    h = jnp.zeros((BW, H), jnp.float32)
    outs_b = [None] * T
    for t in range(T - 1, -1, -1):
        hn = cell(xp_b[t * BW:(t + 1) * BW, :], h, whhT_b, bhh_b)
        h = jnp.where(t < lens, hn, h)
        outs_b[t] = h

    rows = []
    for t in range(T):
        oc = jnp.concatenate([outs_f[t], outs_b[t]], axis=1)
        rows.append(jnp.where(t < lens, oc, 0.0))
    ocat = jnp.concatenate(rows, axis=0)

    proj = jnp.tanh(
        jnp.dot(ocat, wpT_ref[...], preferred_element_type=jnp.float32)
        + bp_ref[...])
    s = jnp.sum(proj * ctx_ref[...], axis=1, keepdims=True)
    s3 = s.reshape(T, BW, 1)
    m = jnp.max(s3, axis=0, keepdims=True)
    e = jnp.exp(s3 - m)
    att = e / jnp.sum(e, axis=0, keepdims=True)
    o3 = ocat.reshape(T, BW, 2 * H) * att
    out_ref[...] = jnp.sum(o3, axis=0)


@functools.partial(jax.jit, static_argnames=("interpret",))
def _char_to_word(padded_char_tensor, sequence_lens, emb, Wih_f, Whh_f,
                  bih_f, bhh_f, Wih_b, Whh_b, bih_b, bhh_b, Wp, bp, ctx,
                  interpret=False):
    NW, T = padded_char_tensor.shape
    V, EMB = emb.shape
    H = Whh_f.shape[1]
    C = Wp.shape[0]
    BW = 256 if NW % 256 == 0 else NW
    n_blocks = NW // BW

    chars3 = padded_char_tensor.astype(jnp.int32).T[:, :, None]
    lens2 = sequence_lens.astype(jnp.int32)[:, None]
    wihT = jnp.concatenate([Wih_f.T, Wih_b.T], axis=1)
    bih = jnp.concatenate([bih_f, bih_b])[None, :]
    out = pl.pallas_call(
        _block_kernel,
        out_shape=jax.ShapeDtypeStruct((NW, 2 * H), jnp.float32),
        grid=(n_blocks,),
        in_specs=[
            pl.BlockSpec((T, BW, 1), lambda i: (0, i, 0)),
            pl.BlockSpec((BW, 1), lambda i: (i, 0)),
            pl.BlockSpec((V, EMB), lambda i: (0, 0)),
            pl.BlockSpec((EMB, 6 * H), lambda i: (0, 0)),
            pl.BlockSpec((H, 3 * H), lambda i: (0, 0)),
            pl.BlockSpec((H, 3 * H), lambda i: (0, 0)),
            pl.BlockSpec((1, 6 * H), lambda i: (0, 0)),
            pl.BlockSpec((1, 3 * H), lambda i: (0, 0)),
            pl.BlockSpec((1, 3 * H), lambda i: (0, 0)),
            pl.BlockSpec((2 * H, C), lambda i: (0, 0)),
            pl.BlockSpec((1, C), lambda i: (0, 0)),
            pl.BlockSpec((1, C), lambda i: (0, 0)),
        ],
        out_specs=pl.BlockSpec((BW, 2 * H), lambda i: (i, 0)),
        compiler_params=pltpu.CompilerParams(
            dimension_semantics=("parallel",),
            vmem_limit_bytes=50 * 1024 * 1024,
        ),
        name="char_to_word",
        interpret=interpret,
    )(
        chars3, lens2, emb, wihT, Whh_f.T, Whh_b.T, bih,
        bhh_f[None, :], bhh_b[None, :], Wp.T, bp[None, :], ctx.T,
    )
    return out


def kernel(padded_char_tensor, sequence_lens, emb, Wih_f, Whh_f, bih_f,
           bhh_f, Wih_b, Whh_b, bih_b, bhh_b, Wp, bp, ctx):
    return _char_to_word(padded_char_tensor, sequence_lens, emb, Wih_f,
                         Whh_f, bih_f, bhh_f, Wih_b, Whh_b, bih_b, bhh_b,
                         Wp, bp, ctx)
